# baseline (speedup 1.0000x reference)
import functools

import jax
import jax.numpy as jnp
from jax import lax
from jax.experimental import pallas as pl
from jax.experimental.pallas import tpu as pltpu
from jax.experimental.pallas import tpu_sc as plsc

_VOCAB = 32
_NEG = -1e9
_POS = 1e9


def kernel(hidden, next_ids):
    del hidden
    B, T = next_ids.shape
    N = B * T
    ids = next_ids.reshape(N).astype(jnp.int32)

    info = plsc.get_sparse_core_info()
    NC, NS, L = info.num_cores, info.num_subcores, info.num_lanes
    NW = NC * NS
    nper = N // NW

    mesh = plsc.VectorSubcoreMesh(core_axis_name="c", subcore_axis_name="s")

    @functools.partial(
        pl.kernel,
        mesh=mesh,
        out_type=jax.ShapeDtypeStruct((N * _VOCAB,), jnp.float32),
        scratch_types=[
            pltpu.VMEM((nper,), jnp.int32),
            pltpu.VMEM((nper * _VOCAB,), jnp.float32),
            pltpu.SemaphoreType.DMA,
            pltpu.SemaphoreType.DMA,
        ],
        compiler_params=pltpu.CompilerParams(needs_layout_passes=False),
    )
    def sc_onehot(ids_hbm, out_hbm, idx_v, buf, sem_in, sem_out):
        wid = lax.axis_index("s") * NC + lax.axis_index("c")
        base = wid * nper
        half = (nper * _VOCAB) // 2

        in_cp = pltpu.async_copy(ids_hbm.at[pl.ds(base, nper)], idx_v, sem_in)

        neg = jnp.full((L,), _NEG, jnp.float32)

        def init_body(i, c):
            for u in range(16):
                buf[pl.ds((i * 16 + u) * L, L)] = neg
            return c

        lax.fori_loop(0, (nper * _VOCAB) // (16 * L), init_body, 0)
        in_cp.wait()

        lane = lax.iota(jnp.int32, L)
        pos = jnp.full((L,), _POS, jnp.float32)

        def scat_body(g, c):
            tok = g * L
            v = idx_v[pl.ds(tok, L)]
            tgt = (v + 1) & (_VOCAB - 1)
            flat = (lane + tok) * _VOCAB + tgt
            plsc.store_scatter(buf, [flat], pos)
            return c

        lax.fori_loop(0, nper // (2 * L), scat_body, 0)
        out_a = pltpu.async_copy(
            buf.at[pl.ds(0, half)],
            out_hbm.at[pl.ds(base * _VOCAB, half)],
            sem_out,
        )
        lax.fori_loop(nper // (2 * L), nper // L, scat_body, 0)
        out_b = pltpu.async_copy(
            buf.at[pl.ds(half, half)],
            out_hbm.at[pl.ds(base * _VOCAB + half, half)],
            sem_out,
        )
        out_a.wait()
        out_b.wait()

    out = sc_onehot(ids)
    return out.reshape(B, T, _VOCAB)
